# Initial kernel scaffold; baseline (speedup 1.0000x reference)
import jax
import jax.numpy as jnp
from jax.experimental import pallas as pl


def kernel(logits, targets):
    raise NotImplementedError("write your pallas kernel here")


# speedup vs baseline: 1.7949x; 1.7949x over previous
import jax
import jax.numpy as jnp
from jax.experimental import pallas as pl
from jax.experimental.pallas import tpu as pltpu

_TOP_K = 5
_IGNORE_INDEX = -100


def _acc_block_kernel(tgt_ref, x_ref, num_ref, den_ref):
    x = x_ref[...]
    tgt = tgt_ref[0]
    R, C = x.shape
    col = jax.lax.broadcasted_iota(jnp.int32, (R, C), 1)
    onehot = col == tgt
    t = jnp.sum(jnp.where(onehot, x, 0.0), axis=1, keepdims=True)
    gt = (x > t).astype(jnp.int32)
    eq_lt = ((x == t) & (col < tgt)).astype(jnp.int32)
    cnt = jnp.sum(gt + eq_lt, axis=1)
    not_ign = tgt[:, 0] != _IGNORE_INDEX
    correct = (cnt < _TOP_K) & not_ign
    num = jnp.sum(correct.astype(jnp.float32))
    den = jnp.sum(not_ign.astype(jnp.float32))

    @pl.when(pl.program_id(0) == 0)
    def _():
        num_ref[0, 0] = 0.0
        den_ref[0, 0] = 0.0

    num_ref[0, 0] += num
    den_ref[0, 0] += den


def kernel(logits, targets):
    B, C = logits.shape
    R = 8
    G = B // R
    tgt3 = targets.astype(jnp.int32).reshape(G, R, 1)
    num, den = pl.pallas_call(
        _acc_block_kernel,
        grid=(G,),
        in_specs=[
            pl.BlockSpec((1, R, 1), lambda i: (i, 0, 0)),
            pl.BlockSpec((R, C), lambda i: (i, 0)),
        ],
        out_specs=[
            pl.BlockSpec(memory_space=pltpu.SMEM, block_shape=(1, 1),
                         index_map=lambda i: (0, 0)),
            pl.BlockSpec(memory_space=pltpu.SMEM, block_shape=(1, 1),
                         index_map=lambda i: (0, 0)),
        ],
        out_shape=[
            jax.ShapeDtypeStruct((1, 1), jnp.float32),
            jax.ShapeDtypeStruct((1, 1), jnp.float32),
        ],
    )(tgt3, logits)
    return num[0, 0] / den[0, 0]


# speedup vs baseline: 2.3100x; 1.2870x over previous
import jax
import jax.numpy as jnp
from jax.experimental import pallas as pl
from jax.experimental.pallas import tpu as pltpu

_TOP_K = 5
_IGNORE_INDEX = -100


def _acc_block_kernel(tgt_ref, x_ref, num_ref, den_ref):
    x = x_ref[...]
    tgt = tgt_ref[0]
    R, C = x.shape
    col = jax.lax.broadcasted_iota(jnp.int32, (R, C), 1)
    onehot = col == tgt
    t = jnp.sum(jnp.where(onehot, x, 0.0), axis=1, keepdims=True)
    gt = (x > t).astype(jnp.int32)
    eq_lt = ((x == t) & (col < tgt)).astype(jnp.int32)
    cnt = jnp.sum(gt + eq_lt, axis=1)
    not_ign = tgt[:, 0] != _IGNORE_INDEX
    correct = (cnt < _TOP_K) & not_ign
    num = jnp.sum(correct.astype(jnp.float32))
    den = jnp.sum(not_ign.astype(jnp.float32))

    @pl.when(pl.program_id(0) == 0)
    def _():
        num_ref[0, 0] = 0.0
        den_ref[0, 0] = 0.0

    num_ref[0, 0] += num
    den_ref[0, 0] += den


def kernel(logits, targets):
    B, C = logits.shape
    R = 32
    G = B // R
    tgt3 = targets.astype(jnp.int32).reshape(G, R, 1)
    num, den = pl.pallas_call(
        _acc_block_kernel,
        grid=(G,),
        in_specs=[
            pl.BlockSpec((1, R, 1), lambda i: (i, 0, 0)),
            pl.BlockSpec((R, C), lambda i: (i, 0)),
        ],
        out_specs=[
            pl.BlockSpec(memory_space=pltpu.SMEM, block_shape=(1, 1),
                         index_map=lambda i: (0, 0)),
            pl.BlockSpec(memory_space=pltpu.SMEM, block_shape=(1, 1),
                         index_map=lambda i: (0, 0)),
        ],
        out_shape=[
            jax.ShapeDtypeStruct((1, 1), jnp.float32),
            jax.ShapeDtypeStruct((1, 1), jnp.float32),
        ],
    )(tgt3, logits)
    return num[0, 0] / den[0, 0]


# speedup vs baseline: 2.3924x; 1.0356x over previous
import jax
import jax.numpy as jnp
from jax.experimental import pallas as pl
from jax.experimental.pallas import tpu as pltpu

_TOP_K = 5
_IGNORE_INDEX = -100


def _acc_block_kernel(tgt_ref, x_ref, num_ref, den_ref):
    x = x_ref[...]
    tgt = tgt_ref[0]
    R, C = x.shape
    col = jax.lax.broadcasted_iota(jnp.int32, (R, C), 1)
    onehot = col == tgt
    t = jnp.sum(jnp.where(onehot, x, 0.0), axis=1, keepdims=True)
    gt = (x > t).astype(jnp.int32)
    eq_lt = ((x == t) & (col < tgt)).astype(jnp.int32)
    cnt = jnp.sum(gt + eq_lt, axis=1)
    not_ign = tgt[:, 0] != _IGNORE_INDEX
    correct = (cnt < _TOP_K) & not_ign
    num = jnp.sum(correct.astype(jnp.float32))
    den = jnp.sum(not_ign.astype(jnp.float32))

    @pl.when(pl.program_id(0) == 0)
    def _():
        num_ref[0, 0] = 0.0
        den_ref[0, 0] = 0.0

    num_ref[0, 0] += num
    den_ref[0, 0] += den


def kernel(logits, targets):
    B, C = logits.shape
    R = 64
    G = B // R
    tgt3 = targets.astype(jnp.int32).reshape(G, R, 1)
    num, den = pl.pallas_call(
        _acc_block_kernel,
        grid=(G,),
        in_specs=[
            pl.BlockSpec((1, R, 1), lambda i: (i, 0, 0)),
            pl.BlockSpec((R, C), lambda i: (i, 0)),
        ],
        out_specs=[
            pl.BlockSpec(memory_space=pltpu.SMEM, block_shape=(1, 1),
                         index_map=lambda i: (0, 0)),
            pl.BlockSpec(memory_space=pltpu.SMEM, block_shape=(1, 1),
                         index_map=lambda i: (0, 0)),
        ],
        out_shape=[
            jax.ShapeDtypeStruct((1, 1), jnp.float32),
            jax.ShapeDtypeStruct((1, 1), jnp.float32),
        ],
    )(tgt3, logits)
    return num[0, 0] / den[0, 0]
